# baseline (speedup 1.0000x reference)
import functools

import numpy as np

import jax
import jax.numpy as jnp
from jax.experimental import pallas as pl
from jax.experimental.shard_map import shard_map
from jax.sharding import Mesh, PartitionSpec as P


def _mm_body(a_ref, b_ref, o_ref):
    o_ref[...] = jnp.dot(a_ref[...], b_ref[...],
                         preferred_element_type=jnp.float32)


def _mm_bias_body(a_ref, b_ref, bias_ref, o_ref, *, relu):
    acc = jnp.dot(a_ref[...], b_ref[...], preferred_element_type=jnp.float32)
    acc = acc + bias_ref[...]
    if relu:
        acc = jnp.maximum(acc, 0.0)
    o_ref[...] = acc


def _matmul(a, b, bias=None, relu=False, block_m=512):
    m, k = a.shape
    k2, n = b.shape
    block_m = min(block_m, m)
    grid = (pl.cdiv(m, block_m),)
    in_specs = [
        pl.BlockSpec((block_m, k), lambda i: (i, 0)),
        pl.BlockSpec((k, n), lambda i: (0, 0)),
    ]
    operands = [a, b]
    if bias is not None:
        in_specs.append(pl.BlockSpec((1, n), lambda i: (0, 0)))
        operands.append(bias.reshape(1, n))
        body = functools.partial(_mm_bias_body, relu=relu)
    else:
        body = _mm_body
    return pl.pallas_call(
        body,
        grid=grid,
        in_specs=in_specs,
        out_specs=pl.BlockSpec((block_m, n), lambda i: (i, 0)),
        out_shape=jax.ShapeDtypeStruct((m, n), jnp.float32),
    )(*operands)


def _net_sharded(x, adj, weights):
    (We1, be1, We2, be2, We3, be3, Wez, bez,
     Wd1, bd1, Wd2, bd2, Wd3, bd3, Wdf, bdf) = weights

    def gcn(h_loc, w, b, relu):
        s_loc = _matmul(h_loc, w)
        s_full = jax.lax.all_gather(s_loc, 'm', axis=0, tiled=True)
        return _matmul(adj, s_full, bias=b, relu=relu)

    h = x
    for w, b in ((We1, be1), (We2, be2), (We3, be3)):
        h = gcn(h, w, b, relu=True)
    z = gcn(h, Wez, bez, relu=False)
    h = z
    for w, b in ((Wd1, bd1), (Wd2, bd2), (Wd3, bd3)):
        h = gcn(h, w, b, relu=True)
    x_recon = gcn(h, Wdf, bdf, relu=False)
    return (z, x_recon)


def _net_single(x, adj, weights):
    (We1, be1, We2, be2, We3, be3, Wez, bez,
     Wd1, bd1, Wd2, bd2, Wd3, bd3, Wdf, bdf) = weights

    def gcn(h, w, b, relu):
        return _matmul(adj, _matmul(h, w), bias=b, relu=relu)

    h = x
    for w, b in ((We1, be1), (We2, be2), (We3, be3)):
        h = gcn(h, w, b, relu=True)
    z = gcn(h, Wez, bez, relu=False)
    h = z
    for w, b in ((Wd1, bd1), (Wd2, bd2), (Wd3, bd3)):
        h = gcn(h, w, b, relu=True)
    x_recon = gcn(h, Wdf, bdf, relu=False)
    return (z, x_recon)


def kernel(x, adj, We1, be1, We2, be2, We3, be3, Wez, bez,
           Wd1, bd1, Wd2, bd2, Wd3, bd3, Wdf, bdf):
    weights = (We1, be1, We2, be2, We3, be3, Wez, bez,
               Wd1, bd1, Wd2, bd2, Wd3, bd3, Wdf, bdf)
    devs = jax.devices()
    if len(devs) >= 2 and x.shape[0] % 2 == 0:
        mesh = Mesh(np.array(devs[:2]), ('m',))
        f = shard_map(
            _net_sharded, mesh=mesh,
            in_specs=(P('m', None), P('m', None),
                      tuple(P() for _ in weights)),
            out_specs=(P('m', None), P('m', None)),
            check_rep=False,
        )
        return f(x, adj, weights)
    return _net_single(x, adj, weights)


# speedup vs baseline: 4.4010x; 4.4010x over previous
import functools

import jax
import jax.numpy as jnp
from jax.experimental import pallas as pl
from jax.experimental.pallas import tpu as pltpu


def _gcn_body(h_ref, w_ref, adj_ref, bias_ref, o_ref, s_ref, *, relu):
    @pl.when(pl.program_id(0) == 0)
    def _():
        s32 = jnp.dot(h_ref[...], w_ref[...],
                      preferred_element_type=jnp.float32)
        s_ref[...] = s32.astype(jnp.bfloat16)

    acc = jnp.dot(adj_ref[...], s_ref[...],
                  preferred_element_type=jnp.float32)
    acc = acc + bias_ref[...]
    if relu:
        acc = jnp.maximum(acc, 0.0)
    o_ref[...] = acc.astype(o_ref.dtype)


def _gcn(h_bf, adj_bf, w_bf, b, relu, out_dtype, block_m=512):
    m, k = h_bf.shape
    k2, n = w_bf.shape
    grid = (pl.cdiv(m, block_m),)
    return pl.pallas_call(
        functools.partial(_gcn_body, relu=relu),
        grid=grid,
        in_specs=[
            pl.BlockSpec((m, k), lambda i: (0, 0)),
            pl.BlockSpec((k, n), lambda i: (0, 0)),
            pl.BlockSpec((block_m, m), lambda i: (i, 0)),
            pl.BlockSpec((1, n), lambda i: (0, 0)),
        ],
        out_specs=pl.BlockSpec((block_m, n), lambda i: (i, 0)),
        out_shape=jax.ShapeDtypeStruct((m, n), out_dtype),
        scratch_shapes=[pltpu.VMEM((m, n), jnp.bfloat16)],
    )(h_bf, w_bf, adj_bf, b.reshape(1, n))


def kernel(x, adj, We1, be1, We2, be2, We3, be3, Wez, bez,
           Wd1, bd1, Wd2, bd2, Wd3, bd3, Wdf, bdf):
    bf = jnp.bfloat16
    adj_bf = adj.astype(bf)
    h = x.astype(bf)
    for w, b in ((We1, be1), (We2, be2), (We3, be3)):
        h = _gcn(h, adj_bf, w.astype(bf), b, relu=True, out_dtype=bf)
    z = _gcn(h, adj_bf, Wez.astype(bf), bez, relu=False,
             out_dtype=jnp.float32)
    h = z.astype(bf)
    for w, b in ((Wd1, bd1), (Wd2, bd2), (Wd3, bd3)):
        h = _gcn(h, adj_bf, w.astype(bf), b, relu=True, out_dtype=bf)
    x_recon = _gcn(h, adj_bf, Wdf.astype(bf), bdf, relu=False,
                   out_dtype=jnp.float32)
    return (z, x_recon)
